# baseline (speedup 1.0000x reference)
import functools

import jax
import jax.numpy as jnp
from jax import lax
from jax.experimental import pallas as pl
from jax.experimental.pallas import tpu as pltpu
from jax.experimental.pallas import tpu_sc as plsc

N = 10000
E = 160000
D = 256
R = 8
NR = N * R

NC = 2
NS = 16
L = 16

NRP = 81920
EPT = E // NS
SEG = 2000
NSEG = EPT // SEG
SVREG = SEG // L

CHUNK = 4096
NPASS = (NRP // NC) // CHUNK
STRIPE = CHUNK // NS
CAP = 2048
BLK = 256
B = 16


def _sc_body(keys_hbm, srcs_hbm, x_hbm, z_hbm, a_hbm, cnt_hbm,
             keys_v, srcs_v, gidx, sidx, blk_l, blk_s, ob_l, ob_s,
             gf, rows, acc, hist, cntv, cntr, zbuf, ploc, psrc, pcnt, sem):
  c = lax.axis_index("c")
  s = lax.axis_index("s")

  pltpu.sync_copy(keys_hbm.at[pl.ds(s * EPT, EPT)], keys_v)
  pltpu.sync_copy(srcs_hbm.at[pl.ds(s * EPT, EPT)], srcs_v)
  pltpu.sync_copy(z_hbm, zbuf)
  plsc.subcore_barrier()

  ones = jnp.ones((L,), jnp.float32)
  lane = lax.iota(jnp.int32, L)
  sent = jnp.full((L,), CHUNK, jnp.int32)
  sc_base = c * (NRP // NC)
  my_lo = s * STRIPE

  def _pass(p, _p):
    base = sc_base + p * CHUNK
    hi = base + CHUNK

    for r0 in range(0, STRIPE, 64):
      pltpu.sync_copy(zbuf, acc.at[pl.ds(r0, 64)])
    for j in range(STRIPE // L):
      hist[pl.ds(j * L, L)] = jnp.zeros((L,), jnp.float32)

    def _seg(g, _g):
      def _scan(i, off):
        kv = keys_v[pl.ds(g * SEG + i * L, L)]
        sv = srcs_v[pl.ds(g * SEG + i * L, L)]
        m = (kv >= base) & (kv < hi)
        loc = kv - base
        mi = m.astype(jnp.int32)
        csum = plsc.cumsum(mi)
        pos = off + csum - mi
        plsc.store_scatter(gidx, [pos], sv, mask=m)
        plsc.store_scatter(sidx, [pos], loc, mask=m)
        return off + jnp.sum(mi)
      off = lax.fori_loop(0, SVREG, _scan, jnp.int32(0))

      nblk = (off + (BLK - 1)) // BLK
      for j in range(BLK // L):
        idxv = off + lane + j * L
        pm = idxv < nblk * BLK
        plsc.store_scatter(sidx, [idxv], sent, mask=pm)

      def _pub(b, _):
        pltpu.sync_copy(gidx.at[pl.ds(b * BLK, BLK)],
                        psrc.at[pl.ds(s * CAP + b * BLK, BLK)])
        pltpu.sync_copy(sidx.at[pl.ds(b * BLK, BLK)],
                        ploc.at[pl.ds(s * CAP + b * BLK, BLK)])
        return 0
      lax.fori_loop(0, nblk, _pub, 0)
      cntv[pl.ds(0, L)] = lane * 0 + off
      pltpu.sync_copy(cntv.at[pl.ds(0, 8)], pcnt.at[pl.ds(s * 8, 8)])
      plsc.subcore_barrier()

      pltpu.sync_copy(pcnt, cntr)
      def _scanner(t, _t):
        tv = plsc.load_gather(cntr, [lane * 0 + t * 8])
        cnt_t = jnp.sum(jnp.where(lane == 0, tv, 0))
        nb = (cnt_t + (BLK - 1)) // BLK
        def _blk(b, _):
          pltpu.sync_copy(ploc.at[pl.ds(t * CAP + b * BLK, BLK)], blk_l)
          pltpu.sync_copy(psrc.at[pl.ds(t * CAP + b * BLK, BLK)], blk_s)
          def _filt(i, moff):
            lv = blk_l[pl.ds(i * L, L)]
            sv = blk_s[pl.ds(i * L, L)]
            m = (lv >= my_lo) & (lv < my_lo + STRIPE)
            ll = lv - my_lo
            mi = m.astype(jnp.int32)
            csum = plsc.cumsum(mi)
            pos = moff + csum - mi
            plsc.store_scatter(ob_l, [pos], ll, mask=m)
            plsc.store_scatter(ob_s, [pos], sv, mask=m)
            plsc.addupdate_scatter(hist, [ll], ones, mask=m)
            return moff + jnp.sum(mi)
          moff = lax.fori_loop(0, BLK // L, _filt, jnp.int32(0))

          nbt = (moff + (B - 1)) // B
          def _fire(b2, _):
            bm = (b2 * B + lane) < moff
            ev = ob_s[pl.ds(b2 * B, B)]
            gf[pl.ds(0, B)] = jnp.where(bm, ev, jnp.zeros((L,), jnp.int32))
            pltpu.async_copy(x_hbm.at[gf], rows, sem).wait()
            llv = ob_l[pl.ds(b2 * B, B)]
            def _addcol(q, _2):
              for k in range(L):
                colv = lane * 0 + (q * L + k)
                v = plsc.load_gather(rows, [lane, colv])
                plsc.addupdate_scatter(acc, [llv, colv], v, mask=bm)
              return 0
            lax.fori_loop(0, D // L, _addcol, 0)
            return 0
          lax.fori_loop(0, nbt, _fire, 0)
          return 0
        lax.fori_loop(0, nb, _blk, 0)
        return 0
      lax.fori_loop(0, NS, _scanner, 0)
      plsc.subcore_barrier()
      return 0
    lax.fori_loop(0, NSEG, _seg, 0)

    pltpu.sync_copy(acc, a_hbm.at[pl.ds(base + s * STRIPE, STRIPE)])
    pltpu.sync_copy(hist, cnt_hbm.at[pl.ds(base + s * STRIPE, STRIPE)])
    return 0
  lax.fori_loop(0, NPASS, _pass, 0)


@functools.partial(jax.jit, static_argnames=())
def _sc_agg(keys, srcs, x):
  mesh = plsc.VectorSubcoreMesh(core_axis_name="c", subcore_axis_name="s",
                                num_cores=NC, num_subcores=NS)
  f = pl.kernel(
      _sc_body,
      out_type=(jax.ShapeDtypeStruct((NRP, D), jnp.float32),
                jax.ShapeDtypeStruct((NRP,), jnp.float32)),
      mesh=mesh,
      compiler_params=pltpu.CompilerParams(needs_layout_passes=False),
      scratch_types=[
          pltpu.VMEM((EPT,), jnp.int32),
          pltpu.VMEM((EPT,), jnp.int32),
          pltpu.VMEM((CAP,), jnp.int32),
          pltpu.VMEM((CAP,), jnp.int32),
          pltpu.VMEM((BLK,), jnp.int32),
          pltpu.VMEM((BLK,), jnp.int32),
          pltpu.VMEM((BLK,), jnp.int32),
          pltpu.VMEM((BLK,), jnp.int32),
          pltpu.VMEM((B,), jnp.int32),
          pltpu.VMEM((B, D), jnp.float32),
          pltpu.VMEM((STRIPE, D), jnp.float32),
          pltpu.VMEM((STRIPE,), jnp.float32),
          pltpu.VMEM((L,), jnp.int32),
          pltpu.VMEM((NS * 8,), jnp.int32),
          pltpu.VMEM_SHARED((64, D), jnp.float32),
          pltpu.VMEM_SHARED((NS * CAP,), jnp.int32),
          pltpu.VMEM_SHARED((NS * CAP,), jnp.int32),
          pltpu.VMEM_SHARED((NS * 8,), jnp.int32),
          pltpu.SemaphoreType.DMA,
      ],
  )
  return f(keys, srcs, x, jnp.zeros((64, D), jnp.float32))


BN = 1000


def _tc_body(relu, a_ref, c_ref, x_ref, wr_ref, ws_ref, o_ref):
  inv = 1.0 / jnp.maximum(c_ref[...], 1.0)
  acc = jnp.dot(x_ref[...], ws_ref[...], preferred_element_type=jnp.float32)
  for r in range(R):
    ar = a_ref[:, r * D:(r + 1) * D] * inv[:, r][:, None]
    acc = acc + jnp.dot(ar, wr_ref[r], preferred_element_type=jnp.float32)
  o_ref[...] = jnp.maximum(acc, 0.0) if relu else acc


def _tc_layer(a, cnt, x, w_rel, w_self, relu):
  grid = (N // BN,)
  return pl.pallas_call(
      functools.partial(_tc_body, relu),
      grid=grid,
      in_specs=[
          pl.BlockSpec((BN, R * D), lambda i: (i, 0)),
          pl.BlockSpec((BN, R), lambda i: (i, 0)),
          pl.BlockSpec((BN, D), lambda i: (i, 0)),
          pl.BlockSpec((R, D, D), lambda i: (0, 0, 0)),
          pl.BlockSpec((D, D), lambda i: (0, 0)),
      ],
      out_specs=pl.BlockSpec((BN, D), lambda i: (i, 0)),
      out_shape=jax.ShapeDtypeStruct((N, D), jnp.float32),
  )(a, cnt, x, w_rel, w_self)


def kernel(X, edge_index, edge_type, W_rel1, W_self1, W_rel2, W_self2):
  src = edge_index[0]
  dst = edge_index[1]
  key = dst * R + edge_type

  a1, cnt = _sc_agg(key, src, X)
  cnt8 = cnt[:NR].reshape(N, R)
  h = _tc_layer(a1[:NR].reshape(N, R * D), cnt8, X,
                W_rel1, W_self1, relu=True)
  a2, _ = _sc_agg(key, src, h)
  out = _tc_layer(a2[:NR].reshape(N, R * D), cnt8, h,
                  W_rel2, W_self2, relu=False)
  return out
